# baseline (speedup 1.0000x reference)
import functools

import jax
import jax.numpy as jnp
from jax import lax
from jax.experimental import pallas as pl
from jax.experimental.pallas import tpu as pltpu
from jax.experimental.pallas import tpu_sc as plsc

D = 768
DFF = 2048
NCL = 4
EPC = 4
NE = NCL * EPC
S = 2048
NA = 4
NASS = S * NA
BLK = 256
NB_MAX = NASS // BLK + NE
NPAD = NB_MAX * BLK
DK = 512
NK = DFF // DK
NEG = -1e30


def _ln(x):
    m = jnp.mean(x, axis=-1, keepdims=True)
    v = jnp.mean((x - m) ** 2, axis=-1, keepdims=True)
    return (x - m) * jax.lax.rsqrt(v + 1e-5)


def _router_body(xf_ref, wc_ref, bc_ref, wea_ref, bea_ref,
                 pos_ref, wslot_ref, cw2_ref, ci2_ref, bemap_ref, nb_ref):
    xf = xf_ref[...]
    cl_logits = jnp.dot(xf, wc_ref[...], preferred_element_type=jnp.float32) + bc_ref[...]
    mx = jnp.max(cl_logits, axis=-1, keepdims=True)
    p = jnp.exp(cl_logits - mx)
    p = p / jnp.sum(p, axis=-1, keepdims=True)
    iota4 = jax.lax.broadcasted_iota(jnp.int32, (S, NCL), 1)
    v1 = jnp.max(p, axis=-1, keepdims=True)
    i1 = jnp.min(jnp.where(p == v1, iota4, NCL), axis=-1, keepdims=True)
    p2 = jnp.where(iota4 == i1, -1.0, p)
    v2 = jnp.max(p2, axis=-1, keepdims=True)
    i2 = jnp.min(jnp.where(p2 == v2, iota4, NCL), axis=-1, keepdims=True)
    csum = v1 + v2
    cw1 = v1 / csum
    cw2 = v2 / csum

    el = jnp.dot(xf, wea_ref[...], preferred_element_type=jnp.float32) + bea_ref[...]
    iota16 = jax.lax.broadcasted_iota(jnp.int32, (S, NE), 1)
    clid16 = iota16 // EPC
    eA = []; eB = []; wA = []; wB = []
    for c in range(NCL):
        mel = jnp.where(clid16 == c, el, NEG)
        ev1 = jnp.max(mel, axis=-1, keepdims=True)
        ei1 = jnp.min(jnp.where(mel == ev1, iota16, NE), axis=-1, keepdims=True)
        mel2 = jnp.where(iota16 == ei1, NEG, mel)
        ev2 = jnp.max(mel2, axis=-1, keepdims=True)
        ei2 = jnp.min(jnp.where(mel2 == ev2, iota16, NE), axis=-1, keepdims=True)
        t = jnp.exp(ev2 - ev1)
        den = 1.0 + t
        eA.append(ei1); eB.append(ei2)
        wA.append(1.0 / den); wB.append(t / den)

    def sel(cslot, lst, zero):
        acc = zero
        for c in range(NCL):
            acc = jnp.where(cslot == c, lst[c], acc)
        return acc

    zi = jnp.zeros((S, 1), jnp.int32)
    zf = jnp.zeros((S, 1), jnp.float32)
    eid0 = sel(i1, eA, zi); eid1 = sel(i1, eB, zi)
    eid2 = sel(i2, eA, zi); eid3 = sel(i2, eB, zi)
    w0 = sel(i1, wA, zf); w1_ = sel(i1, wB, zf)
    w2_ = sel(i2, wA, zf); w3_ = sel(i2, wB, zf)

    eids = [eid0, eid1, eid2, eid3]
    ohs = [(iota16 == e).astype(jnp.float32) for e in eids]
    oh = ohs[0] + ohs[1] + ohs[2] + ohs[3]
    C = oh
    sh = 1
    while sh < S:
        Cs = jnp.concatenate([jnp.zeros((sh, NE), jnp.float32), C[: S - sh, :]], axis=0)
        C = C + Cs
        sh *= 2
    excl = C - oh
    counts = C[S - 1 : S, :]
    padc = jnp.floor((counts + (BLK - 1)) * (1.0 / BLK))
    padcnt = padc * BLK
    r_iota = jax.lax.broadcasted_iota(jnp.int32, (NE, NE), 0)
    c_iota = jax.lax.broadcasted_iota(jnp.int32, (NE, NE), 1)
    tri = (r_iota < c_iota).astype(jnp.float32)
    off = jnp.dot(padcnt, tri, preferred_element_type=jnp.float32)
    pos_cols = []
    for a in range(NA):
        rank = jnp.sum(ohs[a] * excl, axis=-1, keepdims=True)
        offg = jnp.sum(ohs[a] * off, axis=-1, keepdims=True)
        pos_cols.append(rank + offg)
    pos = jnp.concatenate(pos_cols, axis=-1).astype(jnp.int32)
    pos_ref[...] = pos
    wslot_ref[...] = jnp.concatenate([w0, w1_, w2_, w3_], axis=-1)
    cw2_ref[...] = jnp.concatenate([cw1, cw2], axis=-1)
    ci2_ref[...] = jnp.concatenate([i1, i2], axis=-1)

    end_block = (off + padcnt) * (1.0 / BLK)
    brow = jax.lax.broadcasted_iota(jnp.int32, (1, 64), 1).astype(jnp.float32)
    acc = jnp.zeros((1, 64), jnp.float32)
    for e in range(NE):
        acc = acc + (brow >= end_block[0, e]).astype(jnp.float32)
    eidx = jax.lax.broadcasted_iota(jnp.int32, (1, NE), 1).astype(jnp.float32)
    laste = jnp.max(jnp.where(counts > 0.0, eidx, 0.0))
    bemap_ref[...] = jnp.minimum(acc, laste).astype(jnp.int32)
    nb_ref[...] = jnp.sum(padc, axis=-1, keepdims=True).astype(jnp.int32)


def _make_sc_dispatch():
    mesh = plsc.VectorSubcoreMesh(core_axis_name="c", subcore_axis_name="s", num_cores=2, num_subcores=16)
    NW = 32
    RPW = NASS // NW
    CH = 64
    NCH = RPW // CH

    @functools.partial(
        pl.kernel, mesh=mesh,
        out_type=jax.ShapeDtypeStruct((NPAD, D), jnp.float32),
        scratch_types=[
            pltpu.VMEM((NCH, CH), jnp.int32),
            pltpu.VMEM((NCH, CH), jnp.int32),
            pltpu.VMEM((2, CH, D), jnp.float32),
            pltpu.SemaphoreType.DMA,
            pltpu.SemaphoreType.DMA,
            pltpu.SemaphoreType.DMA,
        ],
    )
    def k(xf_hbm, pos_hbm, tok_hbm, xg_hbm, idx_s, tok_s, rows_s, gsem, ssem0, ssem1):
        ssems = [ssem0, ssem1]
        wid = lax.axis_index("s") * 2 + lax.axis_index("c")
        base = wid * RPW
        for ch in range(NCH):
            pltpu.sync_copy(pos_hbm.at[pl.ds(base + ch * CH, CH)], idx_s.at[ch])
            pltpu.sync_copy(tok_hbm.at[pl.ds(base + ch * CH, CH)], tok_s.at[ch])
        scat = [None] * NCH
        for ch in range(NCH):
            if ch >= 2:
                scat[ch - 2].wait()
            buf = rows_s.at[ch % 2]
            pltpu.async_copy(xf_hbm.at[tok_s.at[ch]], buf, gsem).wait()
            scat[ch] = pltpu.async_copy(buf, xg_hbm.at[idx_s.at[ch]], ssems[ch % 2])
        scat[NCH - 2].wait()
        scat[NCH - 1].wait()

    return k


def _make_sc_gather_back():
    mesh = plsc.VectorSubcoreMesh(core_axis_name="c", subcore_axis_name="s", num_cores=2, num_subcores=16)
    NW = 32
    RPW = NASS // NW
    CH = 64
    NCH = RPW // CH

    @functools.partial(
        pl.kernel, mesh=mesh,
        out_type=jax.ShapeDtypeStruct((NASS, D), jnp.float32),
        scratch_types=[
            pltpu.VMEM((NCH, CH), jnp.int32),
            pltpu.VMEM((2, CH, D), jnp.float32),
            pltpu.SemaphoreType.DMA,
            pltpu.SemaphoreType.DMA,
            pltpu.SemaphoreType.DMA,
        ],
    )
    def k(pos_hbm, eog_hbm, eo4_hbm, idx_s, rows_s, gsem, ssem0, ssem1):
        ssems = [ssem0, ssem1]
        wid = lax.axis_index("s") * 2 + lax.axis_index("c")
        base = wid * RPW
        for ch in range(NCH):
            pltpu.sync_copy(pos_hbm.at[pl.ds(base + ch * CH, CH)], idx_s.at[ch])
        wr = [None] * NCH
        for ch in range(NCH):
            if ch >= 2:
                wr[ch - 2].wait()
            buf = rows_s.at[ch % 2]
            pltpu.async_copy(eog_hbm.at[idx_s.at[ch]], buf, gsem).wait()
            wr[ch] = pltpu.async_copy(buf, eo4_hbm.at[pl.ds(base + ch * CH, CH)], ssems[ch % 2])
        wr[NCH - 2].wait()
        wr[NCH - 1].wait()

    return k


def _sc_dispatch(xf, pos_flat):
    tok = (jax.lax.iota(jnp.int32, NASS) // NA).astype(jnp.int32)
    return _make_sc_dispatch()(xf, pos_flat, tok)


def _sc_gather_back(pos_flat, eog):
    return _make_sc_gather_back()(pos_flat, eog)


def _ffn_body(be_ref, xg_ref, w1_ref, w2_ref, w3_ref, gin_ref, bin_ref,
              gout_ref, bout_ref, eog_ref):
    b = pl.program_id(0)
    nb_used = be_ref[NB_MAX]
    valid = b < nb_used
    e = be_ref[b]
    j = e % EPC
    is_swiglu = (j == 0) | (j == 3)
    is_gelu = j == 1

    def finish(hact):
        h = jnp.dot(hact, w3_ref[0], preferred_element_type=jnp.float32)
        y = xg_ref[...] + h
        eog_ref[...] = _ln(y) * gout_ref[0, 0] + bout_ref[0, 0]

    def xnorm():
        return _ln(xg_ref[...]) * gin_ref[0, 0] + bin_ref[0, 0]

    @pl.when(valid & is_swiglu)
    def _():
        xn = xnorm()
        h1 = jnp.dot(xn, w1_ref[0], preferred_element_type=jnp.float32)
        h2 = jnp.dot(xn, w2_ref[0], preferred_element_type=jnp.float32)
        finish(jax.nn.silu(h2) * h2 * h1)

    @pl.when(valid & is_gelu)
    def _():
        h1 = jnp.dot(xnorm(), w1_ref[0], preferred_element_type=jnp.float32)
        finish(0.5 * h1 * (1.0 + jax.lax.erf(h1 * 0.7071067811865476)))

    @pl.when(valid & jnp.logical_not(is_swiglu | is_gelu))
    def _():
        h1 = jnp.dot(xnorm(), w1_ref[0], preferred_element_type=jnp.float32)
        finish(jnp.maximum(h1, 0.0))


def _combine_body(eo4_ref, wslot_ref, cw2_ref, ci2_ref, gcl_ref, bcl_ref, out_ref):
    eo4 = eo4_ref[...]
    w = wslot_ref[...]
    cw = cw2_ref[...]
    ci = ci2_ref[...]
    M = eo4.shape[0]
    iota4 = jax.lax.broadcasted_iota(jnp.int32, (M, NCL), 1)
    out = jnp.zeros((M, D), jnp.float32)
    for kslot in range(2):
        ya = eo4[:, (2 * kslot) * D:(2 * kslot + 1) * D]
        yb = eo4[:, (2 * kslot + 1) * D:(2 * kslot + 2) * D]
        y = w[:, 2 * kslot:2 * kslot + 1] * ya + w[:, 2 * kslot + 1:2 * kslot + 2] * yb
        ohc = (iota4 == ci[:, kslot:kslot + 1]).astype(jnp.float32)
        g = jnp.dot(ohc, gcl_ref[...], preferred_element_type=jnp.float32)
        bb = jnp.dot(ohc, bcl_ref[...], preferred_element_type=jnp.float32)
        cl = _ln(y) * g + bb
        out = out + cw[:, kslot:kslot + 1] * cl
    out_ref[...] = out


def kernel(x, Wc, bc, We, be, W1, W2, W3, g_in, b_in, g_out, b_out, g_cl, b_cl):
    B, Sq, Dq = x.shape
    xf = x.reshape(-1, Dq)
    We_all = We.transpose(1, 0, 2).reshape(D, NE)
    be_all = be.reshape(1, NE)

    pos, wslot, cw2, ci2, bemap, nb = pl.pallas_call(
        _router_body,
        out_shape=[
            jax.ShapeDtypeStruct((S, NA), jnp.int32),
            jax.ShapeDtypeStruct((S, NA), jnp.float32),
            jax.ShapeDtypeStruct((S, 2), jnp.float32),
            jax.ShapeDtypeStruct((S, 2), jnp.int32),
            jax.ShapeDtypeStruct((1, 64), jnp.int32),
            jax.ShapeDtypeStruct((1, 1), jnp.int32),
        ],
    )(xf, Wc, bc.reshape(1, NCL), We_all, be_all)

    be_arr = jnp.concatenate(
        [bemap.reshape(64)[:NB_MAX], nb.reshape(1)]).astype(jnp.int32)
    pos_flat = pos.reshape(NASS)

    xg = _sc_dispatch(xf, pos_flat)

    grid_spec = pltpu.PrefetchScalarGridSpec(
        num_scalar_prefetch=1,
        grid=(NB_MAX,),
        in_specs=[
            pl.BlockSpec((BLK, D), lambda b, be_r: (
                jnp.minimum(b, be_r[NB_MAX] - 1), 0)),
            pl.BlockSpec((1, D, DFF), lambda b, be_r: (be_r[b], 0, 0)),
            pl.BlockSpec((1, D, DFF), lambda b, be_r: (
                jnp.where(be_r[b] % EPC == EPC - 1, be_r[b],
                          (be_r[b] // EPC) * EPC), 0, 0)),
            pl.BlockSpec((1, DFF, D), lambda b, be_r: (be_r[b], 0, 0)),
            pl.BlockSpec((1, 1, D), lambda b, be_r: (be_r[b], 0, 0)),
            pl.BlockSpec((1, 1, D), lambda b, be_r: (be_r[b], 0, 0)),
            pl.BlockSpec((1, 1, D), lambda b, be_r: (be_r[b], 0, 0)),
            pl.BlockSpec((1, 1, D), lambda b, be_r: (be_r[b], 0, 0)),
        ],
        out_specs=pl.BlockSpec((BLK, D), lambda b, be_r: (
            jnp.minimum(b, be_r[NB_MAX] - 1), 0)),
    )
    eog = pl.pallas_call(
        _ffn_body,
        grid_spec=grid_spec,
        out_shape=jax.ShapeDtypeStruct((NPAD, D), jnp.float32),
        compiler_params=pltpu.CompilerParams(
            dimension_semantics=("arbitrary",),
        ),
    )(be_arr, xg, W1, W2, W3,
      g_in.reshape(NE, 1, D), b_in.reshape(NE, 1, D),
      g_out.reshape(NE, 1, D), b_out.reshape(NE, 1, D))

    eo4 = _sc_gather_back(pos_flat, eog)

    M = 512
    out = pl.pallas_call(
        _combine_body,
        grid=(S // M,),
        in_specs=[
            pl.BlockSpec((M, NA * D), lambda m: (m, 0)),
            pl.BlockSpec((M, NA), lambda m: (m, 0)),
            pl.BlockSpec((M, 2), lambda m: (m, 0)),
            pl.BlockSpec((M, 2), lambda m: (m, 0)),
            pl.BlockSpec((NCL, D), lambda m: (0, 0)),
            pl.BlockSpec((NCL, D), lambda m: (0, 0)),
        ],
        out_specs=pl.BlockSpec((M, D), lambda m: (m, 0)),
        out_shape=jax.ShapeDtypeStruct((S, D), jnp.float32),
    )(eo4.reshape(S, NA * D), wslot, cw2, ci2, g_cl, b_cl)

    return out.reshape(B, Sq, Dq)


# speedup vs baseline: 1.0234x; 1.0234x over previous
import functools

import jax
import jax.numpy as jnp
from jax import lax
from jax.experimental import pallas as pl
from jax.experimental.pallas import tpu as pltpu
from jax.experimental.pallas import tpu_sc as plsc

D = 768
DFF = 2048
NCL = 4
EPC = 4
NE = NCL * EPC
S = 2048
NA = 4
NASS = S * NA
BLK = 256
NB_MAX = NASS // BLK + NE
NPAD = NB_MAX * BLK
DK = 512
NK = DFF // DK
NEG = -1e30


def _ln(x):
    m = jnp.mean(x, axis=-1, keepdims=True)
    v = jnp.mean((x - m) ** 2, axis=-1, keepdims=True)
    return (x - m) * jax.lax.rsqrt(v + 1e-5)


def _router_body(xf_ref, wc_ref, bc_ref, wea_ref, bea_ref,
                 pos_ref, wslot_ref, cw2_ref, ci2_ref, bemap_ref, nb_ref):
    xf = xf_ref[...]
    cl_logits = jnp.dot(xf, wc_ref[...], preferred_element_type=jnp.float32) + bc_ref[...]
    mx = jnp.max(cl_logits, axis=-1, keepdims=True)
    p = jnp.exp(cl_logits - mx)
    p = p / jnp.sum(p, axis=-1, keepdims=True)
    iota4 = jax.lax.broadcasted_iota(jnp.int32, (S, NCL), 1)
    v1 = jnp.max(p, axis=-1, keepdims=True)
    i1 = jnp.min(jnp.where(p == v1, iota4, NCL), axis=-1, keepdims=True)
    p2 = jnp.where(iota4 == i1, -1.0, p)
    v2 = jnp.max(p2, axis=-1, keepdims=True)
    i2 = jnp.min(jnp.where(p2 == v2, iota4, NCL), axis=-1, keepdims=True)
    csum = v1 + v2
    cw1 = v1 / csum
    cw2 = v2 / csum

    el = jnp.dot(xf, wea_ref[...], preferred_element_type=jnp.float32) + bea_ref[...]
    iota16 = jax.lax.broadcasted_iota(jnp.int32, (S, NE), 1)
    clid16 = iota16 // EPC
    eA = []; eB = []; wA = []; wB = []
    for c in range(NCL):
        mel = jnp.where(clid16 == c, el, NEG)
        ev1 = jnp.max(mel, axis=-1, keepdims=True)
        ei1 = jnp.min(jnp.where(mel == ev1, iota16, NE), axis=-1, keepdims=True)
        mel2 = jnp.where(iota16 == ei1, NEG, mel)
        ev2 = jnp.max(mel2, axis=-1, keepdims=True)
        ei2 = jnp.min(jnp.where(mel2 == ev2, iota16, NE), axis=-1, keepdims=True)
        t = jnp.exp(ev2 - ev1)
        den = 1.0 + t
        eA.append(ei1); eB.append(ei2)
        wA.append(1.0 / den); wB.append(t / den)

    def sel(cslot, lst, zero):
        acc = zero
        for c in range(NCL):
            acc = jnp.where(cslot == c, lst[c], acc)
        return acc

    zi = jnp.zeros((S, 1), jnp.int32)
    zf = jnp.zeros((S, 1), jnp.float32)
    eid0 = sel(i1, eA, zi); eid1 = sel(i1, eB, zi)
    eid2 = sel(i2, eA, zi); eid3 = sel(i2, eB, zi)
    w0 = sel(i1, wA, zf); w1_ = sel(i1, wB, zf)
    w2_ = sel(i2, wA, zf); w3_ = sel(i2, wB, zf)

    eids = [eid0, eid1, eid2, eid3]
    ohs = [(iota16 == e).astype(jnp.float32) for e in eids]
    oh = ohs[0] + ohs[1] + ohs[2] + ohs[3]
    C = oh
    sh = 1
    while sh < S:
        Cs = jnp.concatenate([jnp.zeros((sh, NE), jnp.float32), C[: S - sh, :]], axis=0)
        C = C + Cs
        sh *= 2
    excl = C - oh
    counts = C[S - 1 : S, :]
    padc = jnp.floor((counts + (BLK - 1)) * (1.0 / BLK))
    padcnt = padc * BLK
    r_iota = jax.lax.broadcasted_iota(jnp.int32, (NE, NE), 0)
    c_iota = jax.lax.broadcasted_iota(jnp.int32, (NE, NE), 1)
    tri = (r_iota < c_iota).astype(jnp.float32)
    off = jnp.dot(padcnt, tri, preferred_element_type=jnp.float32)
    pos_cols = []
    for a in range(NA):
        rank = jnp.sum(ohs[a] * excl, axis=-1, keepdims=True)
        offg = jnp.sum(ohs[a] * off, axis=-1, keepdims=True)
        pos_cols.append(rank + offg)
    pos = jnp.concatenate(pos_cols, axis=-1).astype(jnp.int32)
    pos_ref[...] = pos
    wslot_ref[...] = jnp.concatenate([w0, w1_, w2_, w3_], axis=-1)
    cw2_ref[...] = jnp.concatenate([cw1, cw2], axis=-1)
    ci2_ref[...] = jnp.concatenate([i1, i2], axis=-1)

    end_block = (off + padcnt) * (1.0 / BLK)
    brow = jax.lax.broadcasted_iota(jnp.int32, (1, 64), 1).astype(jnp.float32)
    acc = jnp.zeros((1, 64), jnp.float32)
    for e in range(NE):
        acc = acc + (brow >= end_block[0, e]).astype(jnp.float32)
    eidx = jax.lax.broadcasted_iota(jnp.int32, (1, NE), 1).astype(jnp.float32)
    laste = jnp.max(jnp.where(counts > 0.0, eidx, 0.0))
    bemap_ref[...] = jnp.minimum(acc, laste).astype(jnp.int32)
    nb_ref[...] = jnp.sum(padc, axis=-1, keepdims=True).astype(jnp.int32)


def _make_sc_dispatch():
    mesh = plsc.VectorSubcoreMesh(core_axis_name="c", subcore_axis_name="s", num_cores=2, num_subcores=16)
    NW = 32
    RPW = NASS // NW
    CH = 128
    NCH = RPW // CH

    @functools.partial(
        pl.kernel, mesh=mesh,
        out_type=jax.ShapeDtypeStruct((NPAD, D), jnp.float32),
        scratch_types=[
            pltpu.VMEM((NCH, CH), jnp.int32),
            pltpu.VMEM((CH,), jnp.int32),
            pltpu.VMEM((CH, D), jnp.float32),
            pltpu.SemaphoreType.DMA,
        ],
    )
    def k(xf_hbm, pos_hbm, tok_hbm, xg_hbm, idx_s, tok_s, rows_s, sem):
        wid = lax.axis_index("s") * 2 + lax.axis_index("c")
        base = wid * RPW
        for ch in range(NCH):
            pltpu.sync_copy(pos_hbm.at[pl.ds(base + ch * CH, CH)], idx_s.at[ch])
            pltpu.sync_copy(tok_hbm.at[pl.ds(base + ch * CH, CH)], tok_s)
            pltpu.async_copy(xf_hbm.at[tok_s], rows_s, sem).wait()
            pltpu.async_copy(rows_s, xg_hbm.at[idx_s.at[ch]], sem).wait()

    return k


def _make_sc_gather_back():
    mesh = plsc.VectorSubcoreMesh(core_axis_name="c", subcore_axis_name="s", num_cores=2, num_subcores=16)
    NW = 32
    RPW = NASS // NW
    CH = 128
    NCH = RPW // CH

    @functools.partial(
        pl.kernel, mesh=mesh,
        out_type=jax.ShapeDtypeStruct((NASS, D), jnp.float32),
        scratch_types=[
            pltpu.VMEM((CH,), jnp.int32),
            pltpu.VMEM((CH, D), jnp.float32),
            pltpu.SemaphoreType.DMA,
        ],
    )
    def k(pos_hbm, eog_hbm, eo4_hbm, idx_s, rows_s, sem):
        wid = lax.axis_index("s") * 2 + lax.axis_index("c")
        base = wid * RPW
        for ch in range(NCH):
            pltpu.sync_copy(pos_hbm.at[pl.ds(base + ch * CH, CH)], idx_s)
            pltpu.async_copy(eog_hbm.at[idx_s], rows_s, sem).wait()
            pltpu.sync_copy(rows_s, eo4_hbm.at[pl.ds(base + ch * CH, CH)])

    return k


def _sc_dispatch(xf, pos_flat):
    tok = (jax.lax.iota(jnp.int32, NASS) // NA).astype(jnp.int32)
    return _make_sc_dispatch()(xf, pos_flat, tok)


def _sc_gather_back(pos_flat, eog):
    return _make_sc_gather_back()(pos_flat, eog)


def _ffn_body(be_ref, xg_ref, w1_ref, w2_ref, w3_ref, gin_ref, bin_ref,
              gout_ref, bout_ref, eog_ref):
    b = pl.program_id(0)
    nb_used = be_ref[NB_MAX]
    valid = b < nb_used
    e = be_ref[b]
    j = e % EPC
    is_swiglu = (j == 0) | (j == 3)
    is_gelu = j == 1

    def finish(hact):
        h = jnp.dot(hact, w3_ref[0], preferred_element_type=jnp.float32)
        y = xg_ref[...] + h
        eog_ref[...] = _ln(y) * gout_ref[0, 0] + bout_ref[0, 0]

    def xnorm():
        return _ln(xg_ref[...]) * gin_ref[0, 0] + bin_ref[0, 0]

    @pl.when(valid & is_swiglu)
    def _():
        xn = xnorm()
        h1 = jnp.dot(xn, w1_ref[0], preferred_element_type=jnp.float32)
        h2 = jnp.dot(xn, w2_ref[0], preferred_element_type=jnp.float32)
        finish(jax.nn.silu(h2) * h2 * h1)

    @pl.when(valid & is_gelu)
    def _():
        h1 = jnp.dot(xnorm(), w1_ref[0], preferred_element_type=jnp.float32)
        finish(0.5 * h1 * (1.0 + jax.lax.erf(h1 * 0.7071067811865476)))

    @pl.when(valid & jnp.logical_not(is_swiglu | is_gelu))
    def _():
        h1 = jnp.dot(xnorm(), w1_ref[0], preferred_element_type=jnp.float32)
        finish(jnp.maximum(h1, 0.0))


def _combine_body(eo4_ref, wslot_ref, cw2_ref, ci2_ref, gcl_ref, bcl_ref, out_ref):
    eo4 = eo4_ref[...]
    w = wslot_ref[...]
    cw = cw2_ref[...]
    ci = ci2_ref[...]
    M = eo4.shape[0]
    iota4 = jax.lax.broadcasted_iota(jnp.int32, (M, NCL), 1)
    out = jnp.zeros((M, D), jnp.float32)
    for kslot in range(2):
        ya = eo4[:, (2 * kslot) * D:(2 * kslot + 1) * D]
        yb = eo4[:, (2 * kslot + 1) * D:(2 * kslot + 2) * D]
        y = w[:, 2 * kslot:2 * kslot + 1] * ya + w[:, 2 * kslot + 1:2 * kslot + 2] * yb
        ohc = (iota4 == ci[:, kslot:kslot + 1]).astype(jnp.float32)
        g = jnp.dot(ohc, gcl_ref[...], preferred_element_type=jnp.float32)
        bb = jnp.dot(ohc, bcl_ref[...], preferred_element_type=jnp.float32)
        cl = _ln(y) * g + bb
        out = out + cw[:, kslot:kslot + 1] * cl
    out_ref[...] = out


def kernel(x, Wc, bc, We, be, W1, W2, W3, g_in, b_in, g_out, b_out, g_cl, b_cl):
    B, Sq, Dq = x.shape
    xf = x.reshape(-1, Dq)
    We_all = We.transpose(1, 0, 2).reshape(D, NE)
    be_all = be.reshape(1, NE)

    pos, wslot, cw2, ci2, bemap, nb = pl.pallas_call(
        _router_body,
        out_shape=[
            jax.ShapeDtypeStruct((S, NA), jnp.int32),
            jax.ShapeDtypeStruct((S, NA), jnp.float32),
            jax.ShapeDtypeStruct((S, 2), jnp.float32),
            jax.ShapeDtypeStruct((S, 2), jnp.int32),
            jax.ShapeDtypeStruct((1, 64), jnp.int32),
            jax.ShapeDtypeStruct((1, 1), jnp.int32),
        ],
    )(xf, Wc, bc.reshape(1, NCL), We_all, be_all)

    be_arr = jnp.concatenate(
        [bemap.reshape(64)[:NB_MAX], nb.reshape(1)]).astype(jnp.int32)
    pos_flat = pos.reshape(NASS)

    xg = _sc_dispatch(xf, pos_flat)

    grid_spec = pltpu.PrefetchScalarGridSpec(
        num_scalar_prefetch=1,
        grid=(NB_MAX,),
        in_specs=[
            pl.BlockSpec((BLK, D), lambda b, be_r: (
                jnp.minimum(b, be_r[NB_MAX] - 1), 0)),
            pl.BlockSpec((1, D, DFF), lambda b, be_r: (be_r[b], 0, 0)),
            pl.BlockSpec((1, D, DFF), lambda b, be_r: (
                jnp.where(be_r[b] % EPC == EPC - 1, be_r[b],
                          (be_r[b] // EPC) * EPC), 0, 0)),
            pl.BlockSpec((1, DFF, D), lambda b, be_r: (be_r[b], 0, 0)),
            pl.BlockSpec((1, 1, D), lambda b, be_r: (be_r[b], 0, 0)),
            pl.BlockSpec((1, 1, D), lambda b, be_r: (be_r[b], 0, 0)),
            pl.BlockSpec((1, 1, D), lambda b, be_r: (be_r[b], 0, 0)),
            pl.BlockSpec((1, 1, D), lambda b, be_r: (be_r[b], 0, 0)),
        ],
        out_specs=pl.BlockSpec((BLK, D), lambda b, be_r: (
            jnp.minimum(b, be_r[NB_MAX] - 1), 0)),
    )
    eog = pl.pallas_call(
        _ffn_body,
        grid_spec=grid_spec,
        out_shape=jax.ShapeDtypeStruct((NPAD, D), jnp.float32),
        compiler_params=pltpu.CompilerParams(
            dimension_semantics=("arbitrary",),
        ),
    )(be_arr, xg, W1, W2, W3,
      g_in.reshape(NE, 1, D), b_in.reshape(NE, 1, D),
      g_out.reshape(NE, 1, D), b_out.reshape(NE, 1, D))

    eo4 = _sc_gather_back(pos_flat, eog)

    M = 512
    out = pl.pallas_call(
        _combine_body,
        grid=(S // M,),
        in_specs=[
            pl.BlockSpec((M, NA * D), lambda m: (m, 0)),
            pl.BlockSpec((M, NA), lambda m: (m, 0)),
            pl.BlockSpec((M, 2), lambda m: (m, 0)),
            pl.BlockSpec((M, 2), lambda m: (m, 0)),
            pl.BlockSpec((NCL, D), lambda m: (0, 0)),
            pl.BlockSpec((NCL, D), lambda m: (0, 0)),
        ],
        out_specs=pl.BlockSpec((M, D), lambda m: (m, 0)),
        out_shape=jax.ShapeDtypeStruct((S, D), jnp.float32),
    )(eo4.reshape(S, NA * D), wslot, cw2, ci2, g_cl, b_cl)

    return out.reshape(B, Sq, Dq)
